# baseline (speedup 1.0000x reference)
import functools

import jax
import jax.numpy as jnp
from jax import lax
from jax.experimental import pallas as pl
from jax.experimental.pallas import tpu as pltpu
from jax.experimental.pallas import tpu_sc as plsc

N = 10000
E = 320000
F = 128
H = 4
C = 32
FE = 16
NEG = 0.2

NC = 2
NS = 16
NW = NC * NS
L = 16
EPT = E // NW
G = 48
NCHUNK = EPT // G
GT = EPT - NCHUNK * G
DR = 320
RPT = 632
RLAST = N - (NS - 1) * RPT


def _proj_body(x_ref, wl_ref, bl_ref, wr_ref, br_ref, xl_ref, xr_ref):
    xb = x_ref[...]
    xl_ref[...] = jnp.dot(xb, wl_ref[...], preferred_element_type=jnp.float32) + bl_ref[...]
    xr_ref[...] = jnp.dot(xb, wr_ref[...], preferred_element_type=jnp.float32) + br_ref[...]


def _ee_body(ea_ref, we_ref, ee_ref):
    ee_ref[...] = jnp.dot(ea_ref[...], we_ref[...], preferred_element_type=jnp.float32)


def _final_body(a0_ref, a1_ref, d0_ref, d1_ref, b_ref, x_ref, o_ref):
    acc = a0_ref[...] + a1_ref[...]
    den4 = d0_ref[...] + d1_ref[...]
    heads = lax.broadcasted_iota(jnp.int32, (4, F), 0)
    feats = lax.broadcasted_iota(jnp.int32, (4, F), 1)
    expand = (feats // C == heads).astype(jnp.float32)
    denf = jnp.dot(den4, expand, preferred_element_type=jnp.float32) + 1e-16
    o_ref[...] = jnp.maximum(acc / denf + b_ref[...], 0.0) + x_ref[...]


def _edge_block(xl_b, xr_b, ee_b, al_b, dst_b, attb, iota, ones, hcols, n_ed):
    for g in range(n_ed // L):
        rows = jnp.full((L,), g * L, jnp.int32) + iota
        alphas = []
        for h in range(H):
            def _logit(c, carry):
                col, acc = carry
                av = plsc.load_gather(attb, [col])
                a = plsc.load_gather(xl_b, [rows, col])
                b = plsc.load_gather(xr_b, [rows, col])
                e2 = plsc.load_gather(ee_b, [rows, col])
                z = a + b + e2
                lk = jnp.maximum(z, NEG * z)
                return col + ones, acc + lk * av
            _, acc = pl.loop(
                0, C,
                init_carry=(jnp.full((L,), h * C, jnp.int32),
                            jnp.zeros((L,), jnp.float32)),
                unroll=8,
            )(_logit)
            alphas.append(jnp.exp(acc))
        dstv = dst_b[pl.ds(g * L, L)]
        colb = (dstv & 31) * 4
        for h in range(H):
            plsc.store_scatter(al_b, [rows, colb + hcols[h]], alphas[h])
        for h in range(H):
            al = alphas[h]

            def _msg(c, col):
                v = plsc.load_gather(xl_b, [rows, col])
                plsc.store_scatter(xl_b, [rows, col], v * al)
                return col + ones
            pl.loop(0, C,
                    init_carry=jnp.full((L,), h * C, jnp.int32),
                    unroll=8)(_msg)


def _undo_alpha(al_b, dst_b, iota, hcols, zv, n_ed):
    for g in range(n_ed // L):
        rows = jnp.full((L,), g * L, jnp.int32) + iota
        dstv = dst_b[pl.ds(g * L, L)]
        colb = (dstv & 31) * 4
        for h in range(H):
            plsc.store_scatter(al_b, [rows, colb + hcols[h]], zv)


def _gat_sc_body(xl_hbm, xr_hbm, ee_hbm, src_hbm, dst_hbm, att_hbm,
                 zacc_hbm, zden_hbm,
                 accp_hbm, denp_hbm,
                 xl_b, xr_b, ee_b, al_b, src_b, dst_b, dd_b,
                 srct_b, dstt_b, ddt_b, attb, acc_s, den_s, sem):
    cid = lax.axis_index("c")
    sid = lax.axis_index("s")
    wid = cid * NS + sid
    r0 = sid * RPT

    @pl.when(sid < NS - 1)
    def _():
        pltpu.sync_copy(zacc_hbm.at[pl.ds(r0, RPT)], acc_s.at[pl.ds(r0, RPT)])

    @pl.when(sid == NS - 1)
    def _():
        pltpu.sync_copy(zacc_hbm.at[pl.ds(r0, RLAST)],
                        acc_s.at[pl.ds(r0, RLAST)])

    @pl.when(sid < 2)
    def _():
        hdr = DR // 2
        pltpu.sync_copy(zden_hbm.at[pl.ds(sid * hdr, hdr)],
                        den_s.at[pl.ds(sid * hdr, hdr)])

    pltpu.sync_copy(att_hbm, attb)
    zv = jnp.zeros((L,), jnp.float32)

    @pl.loop(0, G)
    def _zal(j):
        for v in range(F // L):
            al_b[j, pl.ds(v * L, L)] = zv

    plsc.subcore_barrier()

    iota = lax.iota(jnp.int32, L)
    ones = jnp.full((L,), 1, jnp.int32)
    hcols = [jnp.full((L,), h, jnp.int32) for h in range(H)]
    ebase = wid * EPT

    @pl.loop(0, NCHUNK)
    def _chunk(t):
        base = ebase + t * G
        pltpu.sync_copy(src_hbm.at[pl.ds(base, G)], src_b)
        pltpu.sync_copy(dst_hbm.at[pl.ds(base, G)], dst_b)
        d1 = pltpu.async_copy(xl_hbm.at[src_b], xl_b, sem)
        d2 = pltpu.async_copy(xr_hbm.at[dst_b], xr_b, sem)
        pltpu.sync_copy(ee_hbm.at[pl.ds(base, G)], ee_b)
        for g in range(G // L):
            dv = dst_b[pl.ds(g * L, L)]
            dd_b[pl.ds(g * L, L)] = lax.shift_right_logical(dv, 5)
        d1.wait()
        d2.wait()

        _edge_block(xl_b, xr_b, ee_b, al_b, dst_b, attb, iota, ones, hcols, G)

        pltpu.sync_copy(xl_b, acc_s.at[dst_b], add=True)
        pltpu.sync_copy(al_b, den_s.at[dd_b], add=True)
        _undo_alpha(al_b, dst_b, iota, hcols, zv, G)

    tbase = ebase + NCHUNK * G
    pltpu.sync_copy(src_hbm.at[pl.ds(tbase, GT)], srct_b)
    pltpu.sync_copy(dst_hbm.at[pl.ds(tbase, GT)], dstt_b)
    d1 = pltpu.async_copy(xl_hbm.at[srct_b], xl_b.at[pl.ds(0, GT)], sem)
    d2 = pltpu.async_copy(xr_hbm.at[dstt_b], xr_b.at[pl.ds(0, GT)], sem)
    pltpu.sync_copy(ee_hbm.at[pl.ds(tbase, GT)], ee_b.at[pl.ds(0, GT)])
    dv = dstt_b[pl.ds(0, L)]
    ddt_b[pl.ds(0, L)] = lax.shift_right_logical(dv, 5)
    d1.wait()
    d2.wait()
    _edge_block(xl_b, xr_b, ee_b, al_b, dstt_b, attb, iota, ones, hcols, GT)
    pltpu.sync_copy(xl_b.at[pl.ds(0, GT)], acc_s.at[dstt_b], add=True)
    pltpu.sync_copy(al_b.at[pl.ds(0, GT)], den_s.at[ddt_b], add=True)
    _undo_alpha(al_b, dstt_b, iota, hcols, zv, GT)

    plsc.subcore_barrier()

    @pl.when(sid < NS - 1)
    def _():
        pltpu.sync_copy(acc_s.at[pl.ds(r0, RPT)],
                        accp_hbm.at[pl.ds(cid * N + r0, RPT)])

    @pl.when(sid == NS - 1)
    def _():
        pltpu.sync_copy(acc_s.at[pl.ds(r0, RLAST)],
                        accp_hbm.at[pl.ds(cid * N + r0, RLAST)])

    @pl.when(sid < 2)
    def _():
        hdr = DR // 2
        pltpu.sync_copy(den_s.at[pl.ds(sid * hdr, hdr)],
                        denp_hbm.at[pl.ds(cid * DR + sid * hdr, hdr)])


def _build_gat_sc(interpret=False):
    return functools.partial(
        pl.kernel,
        out_type=(jax.ShapeDtypeStruct((2 * N, F), jnp.float32),
                  jax.ShapeDtypeStruct((2 * DR, F), jnp.float32)),
        mesh=plsc.VectorSubcoreMesh(core_axis_name="c", subcore_axis_name="s",
                                    num_cores=NC, num_subcores=NS),
        compiler_params=pltpu.CompilerParams(needs_layout_passes=False),
        interpret=interpret,
        scratch_types=[
            pltpu.VMEM((G, F), jnp.float32),
            pltpu.VMEM((G, F), jnp.float32),
            pltpu.VMEM((G, F), jnp.float32),
            pltpu.VMEM((G, F), jnp.float32),
            pltpu.VMEM((G,), jnp.int32),
            pltpu.VMEM((G,), jnp.int32),
            pltpu.VMEM((G,), jnp.int32),
            pltpu.VMEM((L,), jnp.int32),
            pltpu.VMEM((L,), jnp.int32),
            pltpu.VMEM((L,), jnp.int32),
            pltpu.VMEM((F,), jnp.float32),
            pltpu.VMEM_SHARED((N, F), jnp.float32),
            pltpu.VMEM_SHARED((DR, F), jnp.float32),
            pltpu.SemaphoreType.DMA,
        ],
    )(_gat_sc_body)


_gat_sc = _build_gat_sc()


def kernel(x, edge_index, edge_attr, u, batch, Wl, bl, Wr, br, We, att, bias):
    del u, batch
    xl, xr = pl.pallas_call(
        _proj_body,
        grid=(10,),
        in_specs=[
            pl.BlockSpec((N // 10, F), lambda i: (i, 0)),
            pl.BlockSpec((F, F), lambda i: (0, 0)),
            pl.BlockSpec((1, F), lambda i: (0, 0)),
            pl.BlockSpec((F, F), lambda i: (0, 0)),
            pl.BlockSpec((1, F), lambda i: (0, 0)),
        ],
        out_specs=[
            pl.BlockSpec((N // 10, F), lambda i: (i, 0)),
            pl.BlockSpec((N // 10, F), lambda i: (i, 0)),
        ],
        out_shape=[
            jax.ShapeDtypeStruct((N, F), jnp.float32),
            jax.ShapeDtypeStruct((N, F), jnp.float32),
        ],
    )(x, Wl, bl.reshape(1, F), Wr, br.reshape(1, F))

    EB = 2000
    ee = pl.pallas_call(
        _ee_body,
        grid=(E // EB,),
        in_specs=[
            pl.BlockSpec((EB, FE), lambda i: (i, 0)),
            pl.BlockSpec((FE, F), lambda i: (0, 0)),
        ],
        out_specs=pl.BlockSpec((EB, F), lambda i: (i, 0)),
        out_shape=jax.ShapeDtypeStruct((E, F), jnp.float32),
    )(edge_attr, We)

    src = edge_index[0]
    dst = edge_index[1]
    att_flat = att.reshape(F)
    zacc = jnp.zeros((N, F), jnp.float32)
    zden = jnp.zeros((DR, F), jnp.float32)

    accp, denp = _gat_sc(xl, xr, ee, src, dst, att_flat, zacc, zden)

    d0 = denp[:DR].reshape(-1)[:N * H].reshape(N, H)
    d1 = denp[DR:].reshape(-1)[:N * H].reshape(N, H)

    out = pl.pallas_call(
        _final_body,
        grid=(10,),
        in_specs=[
            pl.BlockSpec((N // 10, F), lambda i: (i, 0)),
            pl.BlockSpec((N // 10, F), lambda i: (i, 0)),
            pl.BlockSpec((N // 10, H), lambda i: (i, 0)),
            pl.BlockSpec((N // 10, H), lambda i: (i, 0)),
            pl.BlockSpec((1, F), lambda i: (0, 0)),
            pl.BlockSpec((N // 10, F), lambda i: (i, 0)),
        ],
        out_specs=pl.BlockSpec((N // 10, F), lambda i: (i, 0)),
        out_shape=jax.ShapeDtypeStruct((N, F), jnp.float32),
    )(accp[:N], accp[N:], d0, d1, bias.reshape(1, F), x)
    return out
